# baseline (speedup 1.0000x reference)
import functools

import jax
import jax.numpy as jnp
from jax import lax
from jax.experimental import pallas as pl
from jax.experimental.pallas import tpu as pltpu
from jax.experimental.pallas import tpu_sc as plsc

_N = 10000
_E = 160000
_D = 256
_C = 64
_NC = 2
_NS = 16
_CH = 128
_EP = 163840
_NP = 10240
_R = 1000


@functools.cache
def _sc_kernels():
    mesh = plsc.VectorSubcoreMesh(core_axis_name="c", subcore_axis_name="s",
                                  num_cores=_NC, num_subcores=_NS)
    deg_k = functools.partial(
        pl.kernel,
        out_type=[jax.ShapeDtypeStruct((_NC * _NP,), jnp.float32),
                  jax.ShapeDtypeStruct((_EP // _CH, _CH), jnp.int32)],
        mesh=mesh,
        scratch_types=[
            pltpu.VMEM((640,), jnp.float32),
            pltpu.VMEM((128,), jnp.float32),
            pltpu.VMEM((40, _CH), jnp.int32),
            pltpu.VMEM((40, _CH), jnp.int32),
            pltpu.VMEM_SHARED((_NP,), jnp.float32),
        ],
    )(_deg_body)
    agg_k = functools.partial(
        pl.kernel,
        out_type=jax.ShapeDtypeStruct((_NC, _N, 128), jnp.float32),
        mesh=mesh,
        scratch_types=[
            pltpu.VMEM((80, _CH), jnp.int32),
            pltpu.VMEM((2, _CH), jnp.int32),
            pltpu.VMEM((2, _CH), jnp.int32),
            pltpu.VMEM((2, _CH, 128), jnp.float32),
            pltpu.VMEM_SHARED((_NP, 128), jnp.float32),
            pltpu.SemaphoreType.DMA,
            pltpu.SemaphoreType.DMA,
        ],
    )(_agg_body)
    return deg_k, agg_k


def _deg_body(src_hbm, dst_hbm, out_hbm, pk_hbm, zv, ones_v, dl, sl, deg_sh):
    c = lax.axis_index("c")
    s = lax.axis_index("s")
    w = c * _NS + s

    pltpu.sync_copy(dst_hbm.at[pl.ds(w * 40, 40)], dl)
    pltpu.sync_copy(src_hbm.at[pl.ds(w * 40, 40)], sl)

    def _zed(i, _):
        zv[pl.ds(i * 16, 16)] = jnp.zeros((16,), jnp.float32)
        return 0
    lax.fori_loop(0, 40, _zed, 0)

    def _one(i, _):
        ones_v[pl.ds(i * 16, 16)] = jnp.ones((16,), jnp.float32)
        return 0
    lax.fori_loop(0, 8, _one, 0)

    pltpu.sync_copy(zv, deg_sh.at[pl.ds(s * 640, 640)])
    plsc.subcore_barrier()

    def _body(k, _):
        pltpu.sync_copy(ones_v, deg_sh.at[dl.at[k]], add=True)
        for i in range(8):
            lane = pl.ds(i * 16, 16)
            dl[k, lane] = (dl[k, lane] << 16) | sl[k, lane]
        return 0
    lax.fori_loop(0, 40, _body, 0)
    pltpu.sync_copy(dl, pk_hbm.at[pl.ds(w * 40, 40)])

    plsc.subcore_barrier()
    pltpu.sync_copy(deg_sh.at[pl.ds(s * 640, 640)],
                    out_hbm.at[pl.ds(c * _NP + s * 640, 640)])


def _agg_body(h2_hbm, pk_hbm, out_hbm, pk, sidx, didx, rows,
              agg_sh, gsem0, gsem1):
    c = lax.axis_index("c")
    s = lax.axis_index("s")
    off = c * _N
    gsems = (gsem0, gsem1)

    pltpu.sync_copy(pk_hbm.at[pl.ds(s * 80, 80)], pk)

    @pl.when(s < 15)
    def _():
        pltpu.sync_copy(h2_hbm.at[pl.ds(off + s * 640, 640)],
                        agg_sh.at[pl.ds(s * 640, 640)])

    @pl.when(s == 15)
    def _():
        pltpu.sync_copy(h2_hbm.at[pl.ds(off + 9600, 400)],
                        agg_sh.at[pl.ds(9600, 400)])
    plsc.subcore_barrier()

    def _unpack_fire(k, b):
        for i in range(8):
            v = pk[k, pl.ds(i * 16, 16)]
            didx[b, pl.ds(i * 16, 16)] = lax.shift_right_logical(v, 16)
            sidx[b, pl.ds(i * 16, 16)] = (v & 0xFFFF) + off
        pltpu.async_copy(h2_hbm.at[sidx.at[b]], rows.at[b], gsems[b])

    _unpack_fire(0, 0)
    _unpack_fire(1, 1)

    def _body(k, _):
        for b in range(2):
            ck = 2 * k + b
            pltpu.make_async_copy(h2_hbm.at[sidx.at[b]], rows.at[b],
                                  gsems[b]).wait()
            pltpu.sync_copy(rows.at[b], agg_sh.at[didx.at[b]], add=True)

            @pl.when(ck + 2 < 80)
            def _():
                _unpack_fire(ck + 2, b)
        return 0
    lax.fori_loop(0, 40, _body, 0)

    plsc.subcore_barrier()

    @pl.when(s < 15)
    def _():
        pltpu.sync_copy(agg_sh.at[pl.ds(s * 640, 640)],
                        out_hbm.at[c, pl.ds(s * 640, 640)])

    @pl.when(s == 15)
    def _():
        pltpu.sync_copy(agg_sh.at[pl.ds(9600, 400)],
                        out_hbm.at[c, pl.ds(9600, 400)])


def _h2_body(x_ref, w_ref, p01_ref, out_ref):
    h = jnp.dot(x_ref[...].astype(jnp.bfloat16),
                w_ref[...].astype(jnp.bfloat16),
                preferred_element_type=jnp.float32)
    deg = 1.0 + p01_ref[:, 0:1] + p01_ref[:, 1:2]
    out_ref[0] = h * lax.rsqrt(deg)


_h2_call = pl.pallas_call(
    _h2_body,
    grid=(_N // _R, 2),
    in_specs=[
        pl.BlockSpec((_R, _D), lambda i, j: (i, 0)),
        pl.BlockSpec((_D, 128), lambda i, j: (0, j)),
        pl.BlockSpec((_R, 2), lambda i, j: (i, 0)),
    ],
    out_specs=pl.BlockSpec((1, _R, 128), lambda i, j: (j, i, 0)),
    out_shape=jax.ShapeDtypeStruct((2, _N, 128), jnp.float32),
)


def _cls_body(agg_ref, p01_ref, wc_ref, bc_ref, bk_ref, out_ref):
    dinv = lax.rsqrt(1.0 + p01_ref[:, 0:1] + p01_ref[:, 1:2])
    e0 = agg_ref[0] * dinv + bc_ref[0:1, 0:128]
    e1 = agg_ref[1] * dinv + bc_ref[0:1, 128:256]
    out_ref[...] = (
        jnp.dot(e0, wc_ref[0:128, :], preferred_element_type=jnp.float32)
        + jnp.dot(e1, wc_ref[128:256, :], preferred_element_type=jnp.float32)
        + bk_ref[...])


_cls_call = pl.pallas_call(
    _cls_body,
    grid=(_N // _R,),
    in_specs=[
        pl.BlockSpec((_NC, _R, 128), lambda i: (0, i, 0)),
        pl.BlockSpec((_R, 2), lambda i: (i, 0)),
        pl.BlockSpec((_D, _C), lambda i: (0, 0)),
        pl.BlockSpec((1, _D), lambda i: (0, 0)),
        pl.BlockSpec((1, _C), lambda i: (0, 0)),
    ],
    out_specs=pl.BlockSpec((_R, _C), lambda i: (i, 0)),
    out_shape=jax.ShapeDtypeStruct((_N, _C), jnp.float32),
)


def kernel(x, edge_index, W_conv, b_conv, W_cls, b_cls):
    src = edge_index[0]
    dst = edge_index[1]
    pad = _EP - _E
    pr = jnp.arange(pad, dtype=jnp.int32)
    src_p = jnp.concatenate([src, pr % _N]).reshape(_EP // _CH, _CH)
    dst_p = jnp.concatenate([dst, _N + pr % (_NP - _N)]).reshape(
        _EP // _CH, _CH)
    deg_k, agg_k = _sc_kernels()
    degraw, pk = deg_k(src_p, dst_p)
    p01 = jnp.stack([degraw[:_N], degraw[_NP:_NP + _N]], axis=1)
    h2 = _h2_call(x, W_conv, p01)
    agg = agg_k(h2.reshape(_NC * _N, 128), pk)
    logits = _cls_call(agg, p01, W_cls,
                       b_conv.reshape(1, _D), b_cls.reshape(1, _C))
    return logits


# speedup vs baseline: 1.0815x; 1.0815x over previous
import functools

import jax
import jax.numpy as jnp
from jax import lax
from jax.experimental import pallas as pl
from jax.experimental.pallas import tpu as pltpu
from jax.experimental.pallas import tpu_sc as plsc

_N = 10000
_E = 160000
_D = 256
_C = 64
_NC = 2
_NS = 16
_CH = 128
_EP = 163840
_NP = 10240
_R = 2000


@functools.cache
def _sc_kernels():
    mesh = plsc.VectorSubcoreMesh(core_axis_name="c", subcore_axis_name="s",
                                  num_cores=_NC, num_subcores=_NS)
    deg_k = functools.partial(
        pl.kernel,
        out_type=[jax.ShapeDtypeStruct((_NC * _NP,), jnp.float32),
                  jax.ShapeDtypeStruct((_EP // _CH, _CH), jnp.int32)],
        mesh=mesh,
        scratch_types=[
            pltpu.VMEM((640,), jnp.float32),
            pltpu.VMEM((128,), jnp.float32),
            pltpu.VMEM((40, _CH), jnp.int32),
            pltpu.VMEM((40, _CH), jnp.int32),
            pltpu.VMEM_SHARED((_NP,), jnp.float32),
        ],
    )(_deg_body)
    agg_k = functools.partial(
        pl.kernel,
        out_type=jax.ShapeDtypeStruct((_NC, _N, 128), jnp.float32),
        mesh=mesh,
        scratch_types=[
            pltpu.VMEM((80, _CH), jnp.int32),
            pltpu.VMEM((2, _CH), jnp.int32),
            pltpu.VMEM((2, _CH), jnp.int32),
            pltpu.VMEM((2, _CH, 128), jnp.float32),
            pltpu.VMEM_SHARED((_NP, 128), jnp.float32),
            pltpu.SemaphoreType.DMA,
            pltpu.SemaphoreType.DMA,
        ],
    )(_agg_body)
    return deg_k, agg_k


def _deg_body(src_hbm, dst_hbm, out_hbm, pk_hbm, zv, ones_v, dl, sl, deg_sh):
    c = lax.axis_index("c")
    s = lax.axis_index("s")
    w = c * _NS + s

    pltpu.sync_copy(dst_hbm.at[pl.ds(w * 40, 40)], dl)
    pltpu.sync_copy(src_hbm.at[pl.ds(w * 40, 40)], sl)

    def _zed(i, _):
        zv[pl.ds(i * 16, 16)] = jnp.zeros((16,), jnp.float32)
        return 0
    lax.fori_loop(0, 40, _zed, 0)

    def _one(i, _):
        ones_v[pl.ds(i * 16, 16)] = jnp.ones((16,), jnp.float32)
        return 0
    lax.fori_loop(0, 8, _one, 0)

    pltpu.sync_copy(zv, deg_sh.at[pl.ds(s * 640, 640)])
    plsc.subcore_barrier()

    def _body(k, _):
        pltpu.sync_copy(ones_v, deg_sh.at[dl.at[k]], add=True)
        for i in range(8):
            lane = pl.ds(i * 16, 16)
            dl[k, lane] = (dl[k, lane] << 16) | sl[k, lane]
        return 0
    lax.fori_loop(0, 40, _body, 0)
    pltpu.sync_copy(dl, pk_hbm.at[pl.ds(w * 40, 40)])

    plsc.subcore_barrier()
    pltpu.sync_copy(deg_sh.at[pl.ds(s * 640, 640)],
                    out_hbm.at[pl.ds(c * _NP + s * 640, 640)])


def _agg_body(h2_hbm, pk_hbm, out_hbm, pk, sidx, didx, rows,
              agg_sh, gsem0, gsem1):
    c = lax.axis_index("c")
    s = lax.axis_index("s")
    off = c * _N
    gsems = (gsem0, gsem1)

    pltpu.sync_copy(pk_hbm.at[pl.ds(s * 80, 80)], pk)

    @pl.when(s < 15)
    def _():
        pltpu.sync_copy(h2_hbm.at[pl.ds(off + s * 640, 640)],
                        agg_sh.at[pl.ds(s * 640, 640)])

    @pl.when(s == 15)
    def _():
        pltpu.sync_copy(h2_hbm.at[pl.ds(off + 9600, 400)],
                        agg_sh.at[pl.ds(9600, 400)])
    plsc.subcore_barrier()

    def _unpack_fire(k, b):
        for i in range(8):
            v = pk[k, pl.ds(i * 16, 16)]
            didx[b, pl.ds(i * 16, 16)] = lax.shift_right_logical(v, 16)
            sidx[b, pl.ds(i * 16, 16)] = (v & 0xFFFF) + off
        pltpu.async_copy(h2_hbm.at[sidx.at[b]], rows.at[b], gsems[b])

    _unpack_fire(0, 0)
    _unpack_fire(1, 1)

    def _body(k, _):
        for b in range(2):
            ck = 2 * k + b
            pltpu.make_async_copy(h2_hbm.at[sidx.at[b]], rows.at[b],
                                  gsems[b]).wait()
            pltpu.sync_copy(rows.at[b], agg_sh.at[didx.at[b]], add=True)

            @pl.when(ck + 2 < 80)
            def _():
                _unpack_fire(ck + 2, b)
        return 0
    lax.fori_loop(0, 40, _body, 0)

    plsc.subcore_barrier()

    @pl.when(s < 15)
    def _():
        pltpu.sync_copy(agg_sh.at[pl.ds(s * 640, 640)],
                        out_hbm.at[c, pl.ds(s * 640, 640)])

    @pl.when(s == 15)
    def _():
        pltpu.sync_copy(agg_sh.at[pl.ds(9600, 400)],
                        out_hbm.at[c, pl.ds(9600, 400)])


def _h2_body(x_ref, w_ref, p01_ref, out_ref):
    h = jnp.dot(x_ref[...].astype(jnp.bfloat16),
                w_ref[...].astype(jnp.bfloat16),
                preferred_element_type=jnp.float32)
    dinv = lax.rsqrt(1.0 + p01_ref[:, 0:1] + p01_ref[:, 1:2])
    out_ref[0] = h[:, 0:128] * dinv
    out_ref[1] = h[:, 128:256] * dinv


_h2_call = pl.pallas_call(
    _h2_body,
    grid=(_N // _R,),
    in_specs=[
        pl.BlockSpec((_R, _D), lambda i: (i, 0)),
        pl.BlockSpec((_D, _D), lambda i: (0, 0)),
        pl.BlockSpec((_R, 2), lambda i: (i, 0)),
    ],
    out_specs=pl.BlockSpec((_NC, _R, 128), lambda i: (0, i, 0)),
    out_shape=jax.ShapeDtypeStruct((2, _N, 128), jnp.float32),
)


def _cls_body(agg_ref, p01_ref, wc_ref, bc_ref, bk_ref, out_ref):
    dinv = lax.rsqrt(1.0 + p01_ref[:, 0:1] + p01_ref[:, 1:2])
    e0 = agg_ref[0] * dinv + bc_ref[0:1, 0:128]
    e1 = agg_ref[1] * dinv + bc_ref[0:1, 128:256]
    out_ref[...] = (
        jnp.dot(e0, wc_ref[0:128, :], preferred_element_type=jnp.float32)
        + jnp.dot(e1, wc_ref[128:256, :], preferred_element_type=jnp.float32)
        + bk_ref[...])


_cls_call = pl.pallas_call(
    _cls_body,
    grid=(_N // _R,),
    in_specs=[
        pl.BlockSpec((_NC, _R, 128), lambda i: (0, i, 0)),
        pl.BlockSpec((_R, 2), lambda i: (i, 0)),
        pl.BlockSpec((_D, _C), lambda i: (0, 0)),
        pl.BlockSpec((1, _D), lambda i: (0, 0)),
        pl.BlockSpec((1, _C), lambda i: (0, 0)),
    ],
    out_specs=pl.BlockSpec((_R, _C), lambda i: (i, 0)),
    out_shape=jax.ShapeDtypeStruct((_N, _C), jnp.float32),
)


def kernel(x, edge_index, W_conv, b_conv, W_cls, b_cls):
    src = edge_index[0]
    dst = edge_index[1]
    pad = _EP - _E
    pr = jnp.arange(pad, dtype=jnp.int32)
    src_p = jnp.concatenate([src, pr % _N]).reshape(_EP // _CH, _CH)
    dst_p = jnp.concatenate([dst, _N + pr % (_NP - _N)]).reshape(
        _EP // _CH, _CH)
    deg_k, agg_k = _sc_kernels()
    degraw, pk = deg_k(src_p, dst_p)
    p01 = jnp.stack([degraw[:_N], degraw[_NP:_NP + _N]], axis=1)
    h2 = _h2_call(x, W_conv, p01)
    agg = agg_k(h2.reshape(_NC * _N, 128), pk)
    logits = _cls_call(agg, p01, W_cls,
                       b_conv.reshape(1, _D), b_cls.reshape(1, _C))
    return logits
